# baseline (speedup 1.0000x reference)
import functools

import jax
import jax.numpy as jnp
from jax import lax
from jax.experimental import pallas as pl
from jax.experimental.pallas import tpu as pltpu
from jax.experimental.pallas import tpu_sc as plsc

NC = 2
NS = 16
NW = NC * NS
LANES = 16

DEG = 32
D = 128
B_PAD = 10240
B_PER_W = B_PAD // NW
CHUNK_DST = 4
CHUNK_ROWS = CHUNK_DST * DEG
N_CHUNKS = B_PER_W // CHUNK_DST
SELF_CHUNK = 40
N_SELF = B_PER_W // SELF_CHUNK


def _sc_gather_body(neigh_hbm, nodes_hbm, table_hbm,
                    self_out, nsum_out,
                    idx2d, sidx, rbuf0, rbuf1, sbuf, sbuf1, oslab,
                    sem0, sem1, ssem):
    wid = lax.axis_index("s") * NC + lax.axis_index("c")

    pltpu.sync_copy(neigh_hbm.at[pl.ds(wid * N_CHUNKS, N_CHUNKS)], idx2d)
    pltpu.sync_copy(nodes_hbm.at[pl.ds(wid * N_SELF, N_SELF)], sidx)

    n_c = D // LANES

    def reduce_chunk(rbuf, g):
        for d in range(CHUNK_DST):
            for c in range(n_c):
                acc = [rbuf[d * DEG + rr, pl.ds(c * LANES, LANES)]
                       for rr in range(DEG)]
                while len(acc) > 1:
                    acc = [acc[i] + acc[i + 1] for i in range(0, len(acc), 2)]
                oslab[g * CHUNK_DST + d, pl.ds(c * LANES, LANES)] = acc[0]

    pltpu.async_copy(table_hbm.at[idx2d.at[0]], rbuf0, sem0)

    def outer(i, carry):
        g0 = 2 * i
        g1 = g0 + 1
        pltpu.async_copy(table_hbm.at[idx2d.at[g1]], rbuf1, sem1)
        pltpu.make_async_copy(table_hbm.at[idx2d.at[g0]], rbuf0, sem0).wait()
        reduce_chunk(rbuf0, g0)

        @pl.when(g0 + 2 < N_CHUNKS)
        def _():
            pltpu.async_copy(table_hbm.at[idx2d.at[g0 + 2]], rbuf0, sem0)

        pltpu.make_async_copy(table_hbm.at[idx2d.at[g1]], rbuf1, sem1).wait()
        reduce_chunk(rbuf1, g1)
        return carry

    lax.fori_loop(0, N_CHUNKS // 2, outer, 0)

    pltpu.sync_copy(oslab, nsum_out.at[pl.ds(wid * B_PER_W, B_PER_W)])

    sb = (sbuf, sbuf1)
    pltpu.async_copy(table_hbm.at[sidx.at[0]], sb[0], ssem)
    for j in range(N_SELF):
        if j + 1 < N_SELF:
            pltpu.async_copy(table_hbm.at[sidx.at[j + 1]], sb[(j + 1) % 2], ssem)
        pltpu.make_async_copy(table_hbm.at[sidx.at[j]], sb[j % 2], ssem).wait()
        pltpu.sync_copy(
            sb[j % 2],
            self_out.at[pl.ds(wid * B_PER_W + j * SELF_CHUNK, SELF_CHUNK)])


@jax.jit
def _sc_gather(neigh2d, nodes2d, table):
    mesh = plsc.VectorSubcoreMesh(core_axis_name="c", subcore_axis_name="s",
                                  num_cores=NC, num_subcores=NS)
    fn = functools.partial(
        pl.kernel,
        out_type=(
            jax.ShapeDtypeStruct((B_PAD, D), jnp.float32),
            jax.ShapeDtypeStruct((B_PAD, D), jnp.float32),
        ),
        mesh=mesh,
        scratch_types=[
            pltpu.VMEM((N_CHUNKS, CHUNK_ROWS), jnp.int32),
            pltpu.VMEM((N_SELF, SELF_CHUNK), jnp.int32),
            pltpu.VMEM((CHUNK_ROWS, D), jnp.float32),
            pltpu.VMEM((CHUNK_ROWS, D), jnp.float32),
            pltpu.VMEM((SELF_CHUNK, D), jnp.float32),
            pltpu.VMEM((SELF_CHUNK, D), jnp.float32),
            pltpu.VMEM((B_PER_W, D), jnp.float32),
            pltpu.SemaphoreType.DMA,
            pltpu.SemaphoreType.DMA,
            pltpu.SemaphoreType.DMA,
        ],
    )(_sc_gather_body)
    return fn(neigh2d, nodes2d, table)


def _mm_body(self_ref, nsum_ref, wa_ref, wb_ref, b_ref, o_ref):
    x = (jnp.dot(self_ref[...], wa_ref[...], preferred_element_type=jnp.float32)
         + jnp.dot(nsum_ref[...], wb_ref[...], preferred_element_type=jnp.float32)
         + b_ref[...])
    o_ref[...] = jnp.maximum(x, 0.0)


def _combine(self_rows, nsum, wa, wb_scaled, b2d):
    blk = 1024
    return pl.pallas_call(
        _mm_body,
        grid=(B_PAD // blk,),
        in_specs=[
            pl.BlockSpec((blk, D), lambda i: (i, 0)),
            pl.BlockSpec((blk, D), lambda i: (i, 0)),
            pl.BlockSpec((D, D), lambda i: (0, 0)),
            pl.BlockSpec((D, D), lambda i: (0, 0)),
            pl.BlockSpec((1, D), lambda i: (0, 0)),
        ],
        out_specs=pl.BlockSpec((blk, D), lambda i: (i, 0)),
        out_shape=jax.ShapeDtypeStruct((B_PAD, D), jnp.float32),
    )(self_rows, nsum, wa, wb_scaled, b2d)


def kernel(nodes, neighbors, table, W1, b1):
    B = nodes.shape[0]
    pad = B_PAD - B
    n_nodes = table.shape[0]
    pad_nodes = (jnp.arange(pad, dtype=jnp.int32) * 131) % n_nodes
    pad_neigh = ((jnp.arange(pad * DEG, dtype=jnp.int32) * 131) % n_nodes
                 ).reshape(pad, DEG)
    nodes_p = jnp.concatenate([nodes, pad_nodes]).reshape(
        B_PAD // SELF_CHUNK, SELF_CHUNK)
    neigh_p = jnp.concatenate([neighbors, pad_neigh], axis=0).reshape(
        B_PAD * DEG // CHUNK_ROWS, CHUNK_ROWS)
    self_rows, nsum = _sc_gather(neigh_p, nodes_p, table)
    wa = W1[:D]
    wb_scaled = W1[D:] * (1.0 / DEG)
    out = _combine(self_rows, nsum, wa, wb_scaled, b1.reshape(1, D))
    return out[:B]


# speedup vs baseline: 1.1442x; 1.1442x over previous
import functools

import jax
import jax.numpy as jnp
from jax import lax
from jax.experimental import pallas as pl
from jax.experimental.pallas import tpu as pltpu
from jax.experimental.pallas import tpu_sc as plsc

NC = 2
NS = 16
NW = NC * NS
LANES = 16

DEG = 32
D = 128
B_PAD = 10240
B_PER_W = B_PAD // NW
CHUNK_DST = 4
CHUNK_ROWS = CHUNK_DST * DEG
N_CHUNKS = B_PER_W // CHUNK_DST
SELF_CHUNK = 40
N_SELF = B_PER_W // SELF_CHUNK


def _sc_gather_body(neigh_hbm, nodes_hbm, table_hbm,
                    self_out, nsum_out,
                    idx2d, sidx, rbuf0, rbuf1, rbuf2, rbuf3,
                    obuf0, obuf1, sbuf, sbuf1,
                    sem0, sem1, sem2, sem3, osem0, osem1, ssem):
    wid = lax.axis_index("s") * NC + lax.axis_index("c")

    pltpu.sync_copy(neigh_hbm.at[pl.ds(wid * N_CHUNKS, N_CHUNKS)], idx2d)
    pltpu.sync_copy(nodes_hbm.at[pl.ds(wid * N_SELF, N_SELF)], sidx)

    n_c = D // LANES

    r_unroll = 8

    def reduce_chunk(rbuf, obuf, orow):
        zero = jnp.zeros((LANES,), jnp.float32)
        def body(r, accs, rbuf=rbuf):
            base = r * r_unroll
            new = []
            for d in range(CHUNK_DST):
                for c in range(n_c):
                    v = accs[d * n_c + c]
                    for rr in range(r_unroll):
                        v = v + rbuf[d * DEG + base + rr, pl.ds(c * LANES, LANES)]
                    new.append(v)
            return tuple(new)
        accs = lax.fori_loop(0, DEG // r_unroll, body,
                             tuple([zero] * (CHUNK_DST * n_c)))
        for d in range(CHUNK_DST):
            for c in range(n_c):
                obuf[orow + d, pl.ds(c * LANES, LANES)] = accs[d * n_c + c]

    bufs = (rbuf0, rbuf1, rbuf2, rbuf3)
    sems = (sem0, sem1, sem2, sem3)
    obufs = (obuf0, obuf1)
    osems = (osem0, osem1)
    for b in range(4):
        pltpu.async_copy(table_hbm.at[idx2d.at[b]], bufs[b], sems[b])

    OG = 2 * CHUNK_DST

    def outer(i, carry):
        for b in range(4):
            c = 4 * i + b
            ob = b // 2
            pltpu.make_async_copy(table_hbm.at[idx2d.at[c]], bufs[b], sems[b]).wait()
            if b % 2 == 0:
                @pl.when(i > 0)
                def _(ob=ob, i=i):
                    off = wid * B_PER_W + (2 * (i - 1) + ob) * OG
                    pltpu.make_async_copy(
                        obufs[ob], nsum_out.at[pl.ds(off, OG)], osems[ob]).wait()
            reduce_chunk(bufs[b], obufs[ob], (b % 2) * CHUNK_DST)
            if b % 2 == 1:
                off = wid * B_PER_W + (2 * i + ob) * OG
                pltpu.async_copy(
                    obufs[ob], nsum_out.at[pl.ds(off, OG)], osems[ob])

            @pl.when(c + 4 < N_CHUNKS)
            def _(b=b, c=c):
                pltpu.async_copy(table_hbm.at[idx2d.at[c + 4]], bufs[b], sems[b])
        return carry

    n_iters = N_CHUNKS // 4
    lax.fori_loop(0, n_iters, outer, 0)

    for ob in range(2):
        off = wid * B_PER_W + (2 * (n_iters - 1) + ob) * OG
        pltpu.make_async_copy(
            obufs[ob], nsum_out.at[pl.ds(off, OG)], osems[ob]).wait()

    sb = (sbuf, sbuf1)
    pltpu.async_copy(table_hbm.at[sidx.at[0]], sb[0], ssem)
    for j in range(N_SELF):
        if j + 1 < N_SELF:
            pltpu.async_copy(table_hbm.at[sidx.at[j + 1]], sb[(j + 1) % 2], ssem)
        pltpu.make_async_copy(table_hbm.at[sidx.at[j]], sb[j % 2], ssem).wait()
        pltpu.sync_copy(
            sb[j % 2],
            self_out.at[pl.ds(wid * B_PER_W + j * SELF_CHUNK, SELF_CHUNK)])


@jax.jit
def _sc_gather(neigh2d, nodes2d, table):
    mesh = plsc.VectorSubcoreMesh(core_axis_name="c", subcore_axis_name="s",
                                  num_cores=NC, num_subcores=NS)
    fn = functools.partial(
        pl.kernel,
        out_type=(
            jax.ShapeDtypeStruct((B_PAD, D), jnp.float32),
            jax.ShapeDtypeStruct((B_PAD, D), jnp.float32),
        ),
        mesh=mesh,
        scratch_types=[
            pltpu.VMEM((N_CHUNKS, CHUNK_ROWS), jnp.int32),
            pltpu.VMEM((N_SELF, SELF_CHUNK), jnp.int32),
            pltpu.VMEM((CHUNK_ROWS, D), jnp.float32),
            pltpu.VMEM((CHUNK_ROWS, D), jnp.float32),
            pltpu.VMEM((CHUNK_ROWS, D), jnp.float32),
            pltpu.VMEM((CHUNK_ROWS, D), jnp.float32),
            pltpu.VMEM((2 * CHUNK_DST, D), jnp.float32),
            pltpu.VMEM((2 * CHUNK_DST, D), jnp.float32),
            pltpu.VMEM((SELF_CHUNK, D), jnp.float32),
            pltpu.VMEM((SELF_CHUNK, D), jnp.float32),
            pltpu.SemaphoreType.DMA,
            pltpu.SemaphoreType.DMA,
            pltpu.SemaphoreType.DMA,
            pltpu.SemaphoreType.DMA,
            pltpu.SemaphoreType.DMA,
            pltpu.SemaphoreType.DMA,
            pltpu.SemaphoreType.DMA,
        ],
    )(_sc_gather_body)
    return fn(neigh2d, nodes2d, table)


def _mm_body(self_ref, nsum_ref, wa_ref, wb_ref, b_ref, o_ref):
    x = (jnp.dot(self_ref[...], wa_ref[...], preferred_element_type=jnp.float32)
         + jnp.dot(nsum_ref[...], wb_ref[...], preferred_element_type=jnp.float32)
         + b_ref[...])
    o_ref[...] = jnp.maximum(x, 0.0)


def _combine(self_rows, nsum, wa, wb_scaled, b2d):
    blk = 1024
    return pl.pallas_call(
        _mm_body,
        grid=(B_PAD // blk,),
        in_specs=[
            pl.BlockSpec((blk, D), lambda i: (i, 0)),
            pl.BlockSpec((blk, D), lambda i: (i, 0)),
            pl.BlockSpec((D, D), lambda i: (0, 0)),
            pl.BlockSpec((D, D), lambda i: (0, 0)),
            pl.BlockSpec((1, D), lambda i: (0, 0)),
        ],
        out_specs=pl.BlockSpec((blk, D), lambda i: (i, 0)),
        out_shape=jax.ShapeDtypeStruct((B_PAD, D), jnp.float32),
    )(self_rows, nsum, wa, wb_scaled, b2d)


def kernel(nodes, neighbors, table, W1, b1):
    B = nodes.shape[0]
    pad = B_PAD - B
    n_nodes = table.shape[0]
    pad_nodes = (jnp.arange(pad, dtype=jnp.int32) * 131) % n_nodes
    pad_neigh = ((jnp.arange(pad * DEG, dtype=jnp.int32) * 131) % n_nodes
                 ).reshape(pad, DEG)
    nodes_p = jnp.concatenate([nodes, pad_nodes]).reshape(
        B_PAD // SELF_CHUNK, SELF_CHUNK)
    neigh_p = jnp.concatenate([neighbors, pad_neigh], axis=0).reshape(
        B_PAD * DEG // CHUNK_ROWS, CHUNK_ROWS)
    self_rows, nsum = _sc_gather(neigh_p, nodes_p, table)
    wa = W1[:D]
    wb_scaled = W1[D:] * (1.0 / DEG)
    out = _combine(self_rows, nsum, wa, wb_scaled, b1.reshape(1, D))
    return out[:B]


# speedup vs baseline: 2.0297x; 1.7739x over previous
import functools

import jax
import jax.numpy as jnp
from jax import lax
from jax.experimental import pallas as pl
from jax.experimental.pallas import tpu as pltpu
from jax.experimental.pallas import tpu_sc as plsc

NC = 2
NS = 16
NW = NC * NS
LANES = 16

DEG = 32
D = 128
B_PAD = 10240
B_PER_W = B_PAD // NW
N_SLICE = 4
SLICE_ROWS = B_PER_W // N_SLICE
SELF_CHUNK = 40
N_SELF = B_PER_W // SELF_CHUNK


def _sc_gather_body(neigh_hbm, nodes_hbm, table_hbm,
                    self_out, nsum_out,
                    idxs, sidx, oslab, sbuf, sbuf1,
                    sem0, sem1, sem2, sem3, ssem):
    wid = lax.axis_index("s") * NC + lax.axis_index("c")

    pltpu.sync_copy(neigh_hbm.at[pl.ds(wid * B_PER_W * DEG, B_PER_W * DEG)],
                    idxs)
    pltpu.sync_copy(nodes_hbm.at[pl.ds(wid * N_SELF, N_SELF)], sidx)

    sems = (sem0, sem1, sem2, sem3)

    def gather(k, b, sem, add):
        src = table_hbm.at[idxs.at[pl.ds(k * B_PER_W + b * SLICE_ROWS,
                                         SLICE_ROWS)]]
        dst = oslab.at[pl.ds(b * SLICE_ROWS, SLICE_ROWS)]
        pltpu.async_copy(src, dst, sem, add=add)

    for b in range(N_SLICE):
        gather(0, b, sems[b], False)

    def outer(k, carry):
        for b in range(N_SLICE):
            src = table_hbm.at[idxs.at[pl.ds(k * B_PER_W + b * SLICE_ROWS,
                                             SLICE_ROWS)]]
            dst = oslab.at[pl.ds(b * SLICE_ROWS, SLICE_ROWS)]
            pltpu.make_async_copy(src, dst, sems[b]).wait()

            @pl.when(k + 1 < DEG)
            def _(k=k, b=b):
                gather(k + 1, b, sems[b], True)
        return carry

    lax.fori_loop(0, DEG, outer, 0)

    pltpu.sync_copy(oslab, nsum_out.at[pl.ds(wid * B_PER_W, B_PER_W)])

    sb = (sbuf, sbuf1)
    pltpu.async_copy(table_hbm.at[sidx.at[0]], sb[0], ssem)
    for j in range(N_SELF):
        if j + 1 < N_SELF:
            pltpu.async_copy(table_hbm.at[sidx.at[j + 1]], sb[(j + 1) % 2], ssem)
        pltpu.make_async_copy(table_hbm.at[sidx.at[j]], sb[j % 2], ssem).wait()
        pltpu.sync_copy(
            sb[j % 2],
            self_out.at[pl.ds(wid * B_PER_W + j * SELF_CHUNK, SELF_CHUNK)])


@jax.jit
def _sc_gather(neigh_flat, nodes2d, table):
    mesh = plsc.VectorSubcoreMesh(core_axis_name="c", subcore_axis_name="s",
                                  num_cores=NC, num_subcores=NS)
    fn = functools.partial(
        pl.kernel,
        out_type=(
            jax.ShapeDtypeStruct((B_PAD, D), jnp.float32),
            jax.ShapeDtypeStruct((B_PAD, D), jnp.float32),
        ),
        mesh=mesh,
        scratch_types=[
            pltpu.VMEM((B_PER_W * DEG,), jnp.int32),
            pltpu.VMEM((N_SELF, SELF_CHUNK), jnp.int32),
            pltpu.VMEM((B_PER_W, D), jnp.float32),
            pltpu.VMEM((SELF_CHUNK, D), jnp.float32),
            pltpu.VMEM((SELF_CHUNK, D), jnp.float32),
            pltpu.SemaphoreType.DMA,
            pltpu.SemaphoreType.DMA,
            pltpu.SemaphoreType.DMA,
            pltpu.SemaphoreType.DMA,
            pltpu.SemaphoreType.DMA,
        ],
    )(_sc_gather_body)
    return fn(neigh_flat, nodes2d, table)


def _mm_body(self_ref, nsum_ref, wa_ref, wb_ref, b_ref, o_ref):
    x = (jnp.dot(self_ref[...], wa_ref[...], preferred_element_type=jnp.float32)
         + jnp.dot(nsum_ref[...], wb_ref[...], preferred_element_type=jnp.float32)
         + b_ref[...])
    o_ref[...] = jnp.maximum(x, 0.0)


def _combine(self_rows, nsum, wa, wb_scaled, b2d):
    blk = 1024
    return pl.pallas_call(
        _mm_body,
        grid=(B_PAD // blk,),
        in_specs=[
            pl.BlockSpec((blk, D), lambda i: (i, 0)),
            pl.BlockSpec((blk, D), lambda i: (i, 0)),
            pl.BlockSpec((D, D), lambda i: (0, 0)),
            pl.BlockSpec((D, D), lambda i: (0, 0)),
            pl.BlockSpec((1, D), lambda i: (0, 0)),
        ],
        out_specs=pl.BlockSpec((blk, D), lambda i: (i, 0)),
        out_shape=jax.ShapeDtypeStruct((B_PAD, D), jnp.float32),
    )(self_rows, nsum, wa, wb_scaled, b2d)


def kernel(nodes, neighbors, table, W1, b1):
    B = nodes.shape[0]
    pad = B_PAD - B
    n_nodes = table.shape[0]
    pad_nodes = (jnp.arange(pad, dtype=jnp.int32) * 131) % n_nodes
    pad_neigh = ((jnp.arange(pad * DEG, dtype=jnp.int32) * 131) % n_nodes
                 ).reshape(pad, DEG)
    nodes_p = jnp.concatenate([nodes, pad_nodes]).reshape(
        B_PAD // SELF_CHUNK, SELF_CHUNK)
    neigh_p = jnp.concatenate([neighbors, pad_neigh], axis=0)
    neigh_flat = neigh_p.reshape(NW, B_PER_W, DEG).transpose(0, 2, 1).reshape(-1)
    self_rows, nsum = _sc_gather(neigh_flat, nodes_p, table)
    wa = W1[:D]
    wb_scaled = W1[D:] * (1.0 / DEG)
    out = _combine(self_rows, nsum, wa, wb_scaled, b1.reshape(1, D))
    return out[:B]


# speedup vs baseline: 2.1958x; 1.0819x over previous
import functools

import jax
import jax.numpy as jnp
from jax import lax
from jax.experimental import pallas as pl
from jax.experimental.pallas import tpu as pltpu
from jax.experimental.pallas import tpu_sc as plsc

NC = 2
NS = 16
NW = NC * NS

DEG = 32
D = 128
B_PAD = 10240
B_PER_W = B_PAD // NW
N_SLICE = 8
SLICE_ROWS = B_PER_W // N_SLICE
N_SELF = 4
SELF_ROWS = B_PER_W // N_SELF


def _sc_gather_body(neigh_hbm, nodes_hbm, table_hbm,
                    self_out, nsum_out,
                    idxs, sidx, oslab, sslab,
                    sem0, sem1, sem2, sem3, sem4, sem5, sem6, sem7, ssem):
    wid = lax.axis_index("s") * NC + lax.axis_index("c")

    pltpu.sync_copy(nodes_hbm.at[pl.ds(wid * B_PER_W, B_PER_W)], sidx)
    pltpu.sync_copy(neigh_hbm.at[pl.ds(wid * B_PER_W * DEG, B_PER_W * DEG)],
                    idxs)

    for j in range(N_SELF):
        pltpu.async_copy(
            table_hbm.at[sidx.at[pl.ds(j * SELF_ROWS, SELF_ROWS)]],
            sslab.at[pl.ds(j * SELF_ROWS, SELF_ROWS)], ssem)

    sems = (sem0, sem1, sem2, sem3, sem4, sem5, sem6, sem7)

    def gather(k, b, add):
        src = table_hbm.at[idxs.at[pl.ds(k * B_PER_W + b * SLICE_ROWS,
                                         SLICE_ROWS)]]
        dst = oslab.at[pl.ds(b * SLICE_ROWS, SLICE_ROWS)]
        pltpu.async_copy(src, dst, sems[b], add=add)

    for b in range(N_SLICE):
        gather(0, b, False)

    def outer(k, carry):
        for b in range(N_SLICE):
            src = table_hbm.at[idxs.at[pl.ds(k * B_PER_W + b * SLICE_ROWS,
                                             SLICE_ROWS)]]
            dst = oslab.at[pl.ds(b * SLICE_ROWS, SLICE_ROWS)]
            pltpu.make_async_copy(src, dst, sems[b]).wait()

            @pl.when(k + 1 < DEG)
            def _(k=k, b=b):
                gather(k + 1, b, True)
        return carry

    lax.fori_loop(0, DEG, outer, 0)

    pltpu.sync_copy(oslab, nsum_out.at[pl.ds(wid * B_PER_W, B_PER_W)])

    for j in range(N_SELF):
        pltpu.make_async_copy(
            table_hbm.at[sidx.at[pl.ds(j * SELF_ROWS, SELF_ROWS)]],
            sslab.at[pl.ds(j * SELF_ROWS, SELF_ROWS)], ssem).wait()
    pltpu.sync_copy(sslab, self_out.at[pl.ds(wid * B_PER_W, B_PER_W)])


@jax.jit
def _sc_gather(neigh_flat, nodes_flat, table):
    mesh = plsc.VectorSubcoreMesh(core_axis_name="c", subcore_axis_name="s",
                                  num_cores=NC, num_subcores=NS)
    fn = functools.partial(
        pl.kernel,
        out_type=(
            jax.ShapeDtypeStruct((B_PAD, D), jnp.float32),
            jax.ShapeDtypeStruct((B_PAD, D), jnp.float32),
        ),
        mesh=mesh,
        scratch_types=[
            pltpu.VMEM((B_PER_W * DEG,), jnp.int32),
            pltpu.VMEM((B_PER_W,), jnp.int32),
            pltpu.VMEM((B_PER_W, D), jnp.float32),
            pltpu.VMEM((B_PER_W, D), jnp.float32),
            pltpu.SemaphoreType.DMA,
            pltpu.SemaphoreType.DMA,
            pltpu.SemaphoreType.DMA,
            pltpu.SemaphoreType.DMA,
            pltpu.SemaphoreType.DMA,
            pltpu.SemaphoreType.DMA,
            pltpu.SemaphoreType.DMA,
            pltpu.SemaphoreType.DMA,
            pltpu.SemaphoreType.DMA,
        ],
    )(_sc_gather_body)
    return fn(neigh_flat, nodes_flat, table)


def _mm_body(self_ref, nsum_ref, wa_ref, wb_ref, b_ref, o_ref):
    x = (jnp.dot(self_ref[...], wa_ref[...], preferred_element_type=jnp.float32)
         + jnp.dot(nsum_ref[...], wb_ref[...], preferred_element_type=jnp.float32)
         + b_ref[...])
    o_ref[...] = jnp.maximum(x, 0.0)


def _combine(self_rows, nsum, wa, wb_scaled, b2d):
    blk = 1024
    return pl.pallas_call(
        _mm_body,
        grid=(B_PAD // blk,),
        in_specs=[
            pl.BlockSpec((blk, D), lambda i: (i, 0)),
            pl.BlockSpec((blk, D), lambda i: (i, 0)),
            pl.BlockSpec((D, D), lambda i: (0, 0)),
            pl.BlockSpec((D, D), lambda i: (0, 0)),
            pl.BlockSpec((1, D), lambda i: (0, 0)),
        ],
        out_specs=pl.BlockSpec((blk, D), lambda i: (i, 0)),
        out_shape=jax.ShapeDtypeStruct((B_PAD, D), jnp.float32),
    )(self_rows, nsum, wa, wb_scaled, b2d)


def kernel(nodes, neighbors, table, W1, b1):
    B = nodes.shape[0]
    pad = B_PAD - B
    n_nodes = table.shape[0]
    pad_nodes = (jnp.arange(pad, dtype=jnp.int32) * 131) % n_nodes
    pad_neigh = ((jnp.arange(pad * DEG, dtype=jnp.int32) * 131) % n_nodes
                 ).reshape(pad, DEG)
    nodes_p = jnp.concatenate([nodes, pad_nodes])
    neigh_p = jnp.concatenate([neighbors, pad_neigh], axis=0)
    neigh_flat = neigh_p.reshape(NW, B_PER_W, DEG).transpose(0, 2, 1).reshape(-1)
    self_rows, nsum = _sc_gather(neigh_flat, nodes_p, table)
    wa = W1[:D]
    wb_scaled = W1[D:] * (1.0 / DEG)
    out = _combine(self_rows, nsum, wa, wb_scaled, b1.reshape(1, D))
    return out[:B]


# speedup vs baseline: 2.2036x; 1.0035x over previous
import functools

import jax
import jax.numpy as jnp
from jax import lax
from jax.experimental import pallas as pl
from jax.experimental.pallas import tpu as pltpu
from jax.experimental.pallas import tpu_sc as plsc

NC = 2
NS = 16
NW = NC * NS

DEG = 32
D = 128
B_PAD = 10240
B_PER_W = B_PAD // NW
N_SLICE = 8
SLICE_ROWS = B_PER_W // N_SLICE
N_SELF = 4
SELF_ROWS = B_PER_W // N_SELF


def _sc_gather_body(neigh_hbm, nodes_hbm, table_hbm,
                    self_out, nsum_out,
                    idxs, sidx, oslab, sslab,
                    sem0, sem1, sem2, sem3, sem4, sem5, sem6, sem7, ssem):
    wid = lax.axis_index("s") * NC + lax.axis_index("c")

    pltpu.sync_copy(nodes_hbm.at[pl.ds(wid * B_PER_W, B_PER_W)], sidx)
    pltpu.sync_copy(neigh_hbm.at[pl.ds(wid * B_PER_W * DEG, B_PER_W * DEG)],
                    idxs)

    for j in range(N_SELF):
        pltpu.async_copy(
            table_hbm.at[sidx.at[pl.ds(j * SELF_ROWS, SELF_ROWS)]],
            sslab.at[pl.ds(j * SELF_ROWS, SELF_ROWS)], ssem)

    sems = (sem0, sem1, sem2, sem3, sem4, sem5, sem6, sem7)

    def gather(k, b, add):
        src = table_hbm.at[idxs.at[pl.ds(k * B_PER_W + b * SLICE_ROWS,
                                         SLICE_ROWS)]]
        dst = oslab.at[pl.ds(b * SLICE_ROWS, SLICE_ROWS)]
        pltpu.async_copy(src, dst, sems[b], add=add)

    for b in range(N_SLICE):
        gather(0, b, False)

    def outer(k, carry):
        for b in range(N_SLICE):
            src = table_hbm.at[idxs.at[pl.ds(k * B_PER_W + b * SLICE_ROWS,
                                             SLICE_ROWS)]]
            dst = oslab.at[pl.ds(b * SLICE_ROWS, SLICE_ROWS)]
            pltpu.make_async_copy(src, dst, sems[b]).wait()

            @pl.when(k + 1 < DEG)
            def _(k=k, b=b):
                gather(k + 1, b, True)
        return carry

    lax.fori_loop(0, DEG, outer, 0)

    pltpu.sync_copy(oslab, nsum_out.at[pl.ds(wid * B_PER_W, B_PER_W)])

    for j in range(N_SELF):
        pltpu.make_async_copy(
            table_hbm.at[sidx.at[pl.ds(j * SELF_ROWS, SELF_ROWS)]],
            sslab.at[pl.ds(j * SELF_ROWS, SELF_ROWS)], ssem).wait()
    pltpu.sync_copy(sslab, self_out.at[pl.ds(wid * B_PER_W, B_PER_W)])


@jax.jit
def _sc_gather(neigh_flat, nodes_flat, table):
    mesh = plsc.VectorSubcoreMesh(core_axis_name="c", subcore_axis_name="s",
                                  num_cores=NC, num_subcores=NS)
    fn = functools.partial(
        pl.kernel,
        out_type=(
            jax.ShapeDtypeStruct((B_PAD, D), jnp.float32),
            jax.ShapeDtypeStruct((B_PAD, D), jnp.float32),
        ),
        mesh=mesh,
        scratch_types=[
            pltpu.VMEM((B_PER_W * DEG,), jnp.int32),
            pltpu.VMEM((B_PER_W,), jnp.int32),
            pltpu.VMEM((B_PER_W, D), jnp.float32),
            pltpu.VMEM((B_PER_W, D), jnp.float32),
        ] + [pltpu.SemaphoreType.DMA] * 9,
    )(_sc_gather_body)
    return fn(neigh_flat, nodes_flat, table)


def _mm_body(self_ref, nsum_ref, wa_ref, wb_ref, b_ref, o_ref):
    x = (jnp.dot(self_ref[...], wa_ref[...], preferred_element_type=jnp.float32)
         + jnp.dot(nsum_ref[...], wb_ref[...], preferred_element_type=jnp.float32)
         + b_ref[...])
    o_ref[...] = jnp.maximum(x, 0.0)


def _combine(self_rows, nsum, wa, wb_scaled, b2d):
    blk = 1024
    return pl.pallas_call(
        _mm_body,
        grid=(B_PAD // blk,),
        in_specs=[
            pl.BlockSpec((blk, D), lambda i: (i, 0)),
            pl.BlockSpec((blk, D), lambda i: (i, 0)),
            pl.BlockSpec((D, D), lambda i: (0, 0)),
            pl.BlockSpec((D, D), lambda i: (0, 0)),
            pl.BlockSpec((1, D), lambda i: (0, 0)),
        ],
        out_specs=pl.BlockSpec((blk, D), lambda i: (i, 0)),
        out_shape=jax.ShapeDtypeStruct((B_PAD, D), jnp.float32),
    )(self_rows, nsum, wa, wb_scaled, b2d)


def kernel(nodes, neighbors, table, W1, b1):
    B = nodes.shape[0]
    pad = B_PAD - B
    n_nodes = table.shape[0]
    pad_nodes = (jnp.arange(pad, dtype=jnp.int32) * 131) % n_nodes
    pad_neigh = ((jnp.arange(pad * DEG, dtype=jnp.int32) * 131) % n_nodes
                 ).reshape(pad, DEG)
    nodes_p = jnp.concatenate([nodes, pad_nodes])
    neigh_p = jnp.concatenate([neighbors, pad_neigh], axis=0)
    neigh_flat = neigh_p.reshape(NW, B_PER_W, DEG).transpose(0, 2, 1).reshape(-1)
    self_rows, nsum = _sc_gather(neigh_flat, nodes_p, table)
    wa = W1[:D]
    wb_scaled = W1[D:] * (1.0 / DEG)
    out = _combine(self_rows, nsum, wa, wb_scaled, b1.reshape(1, D))
    return out[:B]
